# baseline (speedup 1.0000x reference)
import jax
import jax.numpy as jnp
from jax import lax
from jax.experimental import pallas as pl
from jax.experimental.pallas import tpu as pltpu
from jax.experimental.pallas import tpu_sc as plsc

H = 384
W = 384
HALF = 256
D = 2 * HALF

NC = 2
NS = 16
NW = NC * NS
NCG = 4
NRG = NW // NCG
RPW = H // NRG
JC = W // NCG
NLANE = 16
NV = HALF // NLANE

_MESH = plsc.VectorSubcoreMesh(core_axis_name="c", subcore_axis_name="s")


def _sc_body(
    row_hbm, col_hbm, out_hbm, rowstage, buf_a, buf_b, sem_a, sem_b, sem_c, sem_d
):
    cid = lax.axis_index("c")
    sid = lax.axis_index("s")
    wid = sid * NC + cid
    rg = wid // NCG
    cc = wid % NCG
    row0 = rg * RPW
    j0 = cc * JC
    off = pl.multiple_of(row0 * HALF, 8)
    pltpu.sync_copy(row_hbm.at[pl.ds(off, RPW * HALF)], rowstage)
    bufs = (buf_a, buf_b)
    sems = ((sem_a, sem_b), (sem_c, sem_d))
    for k in (0, 1):
        pltpu.sync_copy(
            col_hbm.at[pl.ds(j0, JC)], bufs[k].at[:, pl.ds(HALF, HALF)]
        )
    pend = [None, None]
    for i in range(RPW):
        k = i % 2
        buf = bufs[k]
        if pend[k] is not None:
            for cp in pend[k]:
                cp.wait()
        regs = [
            rowstage[pl.ds(i * HALF + v * NLANE, NLANE)] for v in range(NV)
        ]

        def fill(j, carry, _buf=buf, _regs=regs):
            for v in range(NV):
                _buf[j, pl.ds(v * NLANE, NLANE)] = _regs[v]
            return carry

        lax.fori_loop(0, JC, fill, 0)
        half_j = JC // 2
        pend[k] = (
            pltpu.async_copy(
                buf.at[pl.ds(0, half_j)],
                out_hbm.at[row0 + i, pl.ds(j0, half_j)],
                sems[k][0],
            ),
            pltpu.async_copy(
                buf.at[pl.ds(half_j, half_j)],
                out_hbm.at[row0 + i, pl.ds(j0 + half_j, half_j)],
                sems[k][1],
            ),
        )
    for k in (0, 1):
        if pend[k] is not None:
            for cp in pend[k]:
                cp.wait()


def kernel(row_embed, col_embed, h, w):
    del h, w
    run = pl.kernel(
        _sc_body,
        out_type=jax.ShapeDtypeStruct((H, W, D), jnp.float32),
        mesh=_MESH,
        scratch_types=[
            pltpu.VMEM((RPW * HALF,), jnp.float32),
            pltpu.VMEM((JC, D), jnp.float32),
            pltpu.VMEM((JC, D), jnp.float32),
            pltpu.SemaphoreType.DMA,
            pltpu.SemaphoreType.DMA,
            pltpu.SemaphoreType.DMA,
            pltpu.SemaphoreType.DMA,
        ],
    )
    return run(row_embed.reshape(-1), col_embed)


# speedup vs baseline: 1.0026x; 1.0026x over previous
import jax
import jax.numpy as jnp
from jax import lax
from jax.experimental import pallas as pl
from jax.experimental.pallas import tpu as pltpu
from jax.experimental.pallas import tpu_sc as plsc

H = 384
W = 384
HALF = 256
D = 2 * HALF

NC = 2
NS = 16
NW = NC * NS
NCG = 4
NRG = NW // NCG
RPW = H // NRG
JC = W // NCG
NLANE = 16
NV = HALF // NLANE

_MESH = plsc.VectorSubcoreMesh(core_axis_name="c", subcore_axis_name="s")


def _sc_body(row_hbm, col_hbm, out_hbm, rowstage, buf_a, buf_b, sem_a, sem_b):
    cid = lax.axis_index("c")
    sid = lax.axis_index("s")
    wid = sid * NC + cid
    rg = wid // NCG
    cc = wid % NCG
    row0 = rg * RPW
    j0 = cc * JC
    off = pl.multiple_of(row0 * HALF, 8)
    pltpu.sync_copy(row_hbm.at[pl.ds(off, RPW * HALF)], rowstage)
    bufs = (buf_a, buf_b)
    sems = (sem_a, sem_b)
    for k in (0, 1):
        pltpu.sync_copy(
            col_hbm.at[pl.ds(j0, JC)], bufs[k].at[:, pl.ds(HALF, HALF)]
        )
    pend = [None, None]
    for i in range(RPW):
        k = i % 2
        buf = bufs[k]
        if pend[k] is not None:
            pend[k].wait()
        regs = [
            rowstage[pl.ds(i * HALF + v * NLANE, NLANE)] for v in range(NV)
        ]

        def fill(j, carry, _buf=buf, _regs=regs):
            for v in range(NV):
                _buf[j, pl.ds(v * NLANE, NLANE)] = _regs[v]
            return carry

        lax.fori_loop(0, JC, fill, 0)
        pend[k] = pltpu.async_copy(
            buf, out_hbm.at[row0 + i, pl.ds(j0, JC)], sems[k]
        )
    for k in (0, 1):
        if pend[k] is not None:
            pend[k].wait()


def kernel(row_embed, col_embed, h, w):
    del h, w
    run = pl.kernel(
        _sc_body,
        out_type=jax.ShapeDtypeStruct((H, W, D), jnp.float32),
        mesh=_MESH,
        scratch_types=[
            pltpu.VMEM((RPW * HALF,), jnp.float32),
            pltpu.VMEM((JC, D), jnp.float32),
            pltpu.VMEM((JC, D), jnp.float32),
            pltpu.SemaphoreType.DMA,
            pltpu.SemaphoreType.DMA,
        ],
    )
    return run(row_embed.reshape(-1), col_embed)


# speedup vs baseline: 1.0055x; 1.0028x over previous
import jax
import jax.numpy as jnp
from jax import lax
from jax.experimental import pallas as pl
from jax.experimental.pallas import tpu as pltpu
from jax.experimental.pallas import tpu_sc as plsc

H = 384
W = 384
HALF = 256
D = 2 * HALF

NC = 2
NS = 16
NW = NC * NS
NCG = 4
NRG = NW // NCG
RPW = H // NRG
JC = W // NCG
NLANE = 16
NV = HALF // NLANE

_MESH = plsc.VectorSubcoreMesh(core_axis_name="c", subcore_axis_name="s")


def _sc_body(row_hbm, col_hbm, out_hbm, rowstage, buf_a, buf_b, sem_a, sem_b):
    cid = lax.axis_index("c")
    sid = lax.axis_index("s")
    wid = sid * NC + cid
    rg = wid // NCG
    cc = wid % NCG
    row0 = rg * RPW
    j0 = cc * JC
    pltpu.sync_copy(row_hbm.at[pl.ds(row0, RPW)], rowstage)
    bufs = (buf_a, buf_b)
    sems = (sem_a, sem_b)
    for k in (0, 1):
        pltpu.sync_copy(
            col_hbm.at[pl.ds(j0, JC)], bufs[k].at[:, pl.ds(HALF, HALF)]
        )
    pend = [None, None]
    for i in range(RPW):
        k = i % 2
        buf = bufs[k]
        if pend[k] is not None:
            pend[k].wait()
        regs = [
            rowstage[i, pl.ds(v * NLANE, NLANE)] for v in range(NV)
        ]

        def fill(j, carry, _buf=buf, _regs=regs):
            for v in range(NV):
                _buf[j, pl.ds(v * NLANE, NLANE)] = _regs[v]
            return carry

        lax.fori_loop(0, JC, fill, 0)
        pend[k] = pltpu.async_copy(
            buf, out_hbm.at[row0 + i, pl.ds(j0, JC)], sems[k]
        )
    for k in (0, 1):
        if pend[k] is not None:
            pend[k].wait()


def kernel(row_embed, col_embed, h, w):
    del h, w
    run = pl.kernel(
        _sc_body,
        out_type=jax.ShapeDtypeStruct((H, W, D), jnp.float32),
        mesh=_MESH,
        scratch_types=[
            pltpu.VMEM((RPW, HALF), jnp.float32),
            pltpu.VMEM((JC, D), jnp.float32),
            pltpu.VMEM((JC, D), jnp.float32),
            pltpu.SemaphoreType.DMA,
            pltpu.SemaphoreType.DMA,
        ],
    )
    return run(row_embed, col_embed)
